# baseline (speedup 1.0000x reference)
import functools

import jax
import jax.numpy as jnp
from jax import lax
from jax.experimental import pallas as pl
from jax.experimental.pallas import tpu as pltpu
from jax.experimental.pallas import tpu_sc as plsc

_NUM_IN = 20
_NUM_OUT = 64
_BATCH = 16384
_NW = 32
_BPW = _BATCH // _NW
_GRP = 16
_NGRP = _BPW // _GRP
_ELEMS = _BPW * _NUM_OUT
_ROW = 128
_NROW = _ELEMS // _ROW


def _sc_body(x_hbm, lut_hbm, out_hbm, xv, idxl, dst, sem):
    wid = lax.axis_index("s") * 2 + lax.axis_index("c")
    base = wid * _BPW
    pltpu.sync_copy(x_hbm.at[pl.ds(base * _NUM_IN, _BPW * _NUM_IN)], xv)

    iota = lax.iota(jnp.int32, _GRP)

    def grp(g, carry):
        row_base = (g * _GRP + iota) * _NUM_IN
        acc = jnp.zeros((_GRP,), jnp.int32)
        for i in range(_NUM_IN):
            v = plsc.load_gather(xv, [row_base + i])
            acc = acc + jnp.where(v > 0.0, jnp.int32(1 << i), jnp.int32(0))
        addr0 = ((acc >> 7) << 10) + (acc & 127)
        q = g >> 3
        col = (g & 7) * _GRP
        for c in range(_NUM_OUT):
            off = (c // 8) * 8388608 + (c % 8) * 128
            idxl[pl.ds((q * 64 + c) * _ROW + col, _GRP)] = addr0 + off

        @pl.when((g & 7) == 7)
        def _fire():
            pltpu.async_copy(lut_hbm.at[idxl.at[pl.ds(q * 8192, 8192)]],
                             dst.at[pl.ds(q * 8192, 8192)], sem)

        return carry

    lax.fori_loop(0, _NGRP, grp, 0)
    pltpu.make_async_copy(lut_hbm.at[pl.ds(0, _ELEMS)], dst, sem).wait()
    for q in range(4):
        for cb8 in range(8):
            pltpu.sync_copy(
                dst.at[pl.ds((q * 8 + cb8) * 1024, 1024)],
                out_hbm.at[pl.ds(cb8 * 131072 + (4 * wid + q) * 1024,
                                 1024)])


@functools.partial(jax.jit, static_argnames=())
def kernel(x, lut):
    mesh = plsc.VectorSubcoreMesh(core_axis_name="c", subcore_axis_name="s")
    run = pl.kernel(
        _sc_body,
        out_type=jax.ShapeDtypeStruct((_BATCH * _NUM_OUT,), jnp.float32),
        mesh=mesh,
        scratch_types=(
            pltpu.VMEM((_BPW * _NUM_IN,), jnp.float32),
            pltpu.VMEM((_ELEMS,), jnp.int32),
            pltpu.VMEM((_ELEMS,), jnp.float32),
            pltpu.SemaphoreType.DMA,
        ),
        compiler_params=pltpu.CompilerParams(
            needs_layout_passes=False, use_tc_tiling_on_sc=False),
    )
    lut_flat = (
        lut.T.reshape(8, 8, 8192, 128).transpose(0, 2, 1, 3).reshape(-1)
    )
    out = run(x.reshape(-1), lut_flat)
    return out.reshape(8, 128, 8, 128).transpose(0, 2, 1, 3).reshape(
        _NUM_OUT, _BATCH).T


# speedup vs baseline: 1.0260x; 1.0260x over previous
import functools

import jax
import jax.numpy as jnp
from jax import lax
from jax.experimental import pallas as pl
from jax.experimental.pallas import tpu as pltpu
from jax.experimental.pallas import tpu_sc as plsc

_NUM_IN = 20
_NUM_OUT = 64
_BATCH = 16384
_NW = 32
_BPW = _BATCH // _NW
_GRP = 16
_NGRP = _BPW // _GRP
_ELEMS = _BPW * _NUM_OUT
_ROW = 128
_NROW = _ELEMS // _ROW


def _sc_body(x_hbm, lut_hbm, out_hbm, xv, idxl, dst,
             sg0, sg1, sg2, sg3, so):
    wid = lax.axis_index("s") * 2 + lax.axis_index("c")
    base = wid * _BPW
    pltpu.sync_copy(x_hbm.at[pl.ds(base * _NUM_IN, _BPW * _NUM_IN)], xv)

    iota = lax.iota(jnp.int32, _GRP)

    def grp(g, carry):
        row_base = (g * _GRP + iota) * _NUM_IN
        acc = jnp.zeros((_GRP,), jnp.int32)
        for i in range(_NUM_IN):
            v = plsc.load_gather(xv, [row_base + i])
            acc = acc + jnp.where(v > 0.0, jnp.int32(1 << i), jnp.int32(0))
        addr0 = ((acc >> 7) << 10) + (acc & 127)
        q = g >> 3
        col = (g & 7) * _GRP
        for c in range(_NUM_OUT):
            off = (c // 8) * 8388608 + (c % 8) * 128
            idxl[pl.ds((q * 64 + c) * _ROW + col, _GRP)] = addr0 + off

        sgs = (sg0, sg1, sg2, sg3)
        for qq in range(4):

            @pl.when(g == qq * 8 + 7)
            def _fire(qq=qq):
                pltpu.async_copy(
                    lut_hbm.at[idxl.at[pl.ds(qq * 8192, 8192)]],
                    dst.at[pl.ds(qq * 8192, 8192)], sgs[qq])

        return carry

    lax.fori_loop(0, _NGRP, grp, 0)
    for q, sg in enumerate((sg0, sg1, sg2, sg3)):
        pltpu.make_async_copy(lut_hbm.at[pl.ds(0, 8192)],
                              dst.at[pl.ds(q * 8192, 8192)], sg).wait()
        for cb8 in range(8):
            pltpu.async_copy(
                dst.at[pl.ds((q * 8 + cb8) * 1024, 1024)],
                out_hbm.at[pl.ds(cb8 * 131072 + (4 * wid + q) * 1024,
                                 1024)], so)
    pltpu.make_async_copy(dst, out_hbm.at[pl.ds(0, _ELEMS)], so).wait()


@functools.partial(jax.jit, static_argnames=())
def kernel(x, lut):
    mesh = plsc.VectorSubcoreMesh(core_axis_name="c", subcore_axis_name="s")
    run = pl.kernel(
        _sc_body,
        out_type=jax.ShapeDtypeStruct((_BATCH * _NUM_OUT,), jnp.float32),
        mesh=mesh,
        scratch_types=(
            pltpu.VMEM((_BPW * _NUM_IN,), jnp.float32),
            pltpu.VMEM((_ELEMS,), jnp.int32),
            pltpu.VMEM((_ELEMS,), jnp.float32),
            pltpu.SemaphoreType.DMA,
            pltpu.SemaphoreType.DMA,
            pltpu.SemaphoreType.DMA,
            pltpu.SemaphoreType.DMA,
            pltpu.SemaphoreType.DMA,
        ),
        compiler_params=pltpu.CompilerParams(
            needs_layout_passes=False, use_tc_tiling_on_sc=False),
    )
    lut_flat = (
        lut.T.reshape(8, 8, 8192, 128).transpose(0, 2, 1, 3).reshape(-1)
    )
    out = run(x.reshape(-1), lut_flat)
    return out.reshape(8, 128, 8, 128).transpose(0, 2, 1, 3).reshape(
        _NUM_OUT, _BATCH).T


# speedup vs baseline: 1.2319x; 1.2007x over previous
import functools

import jax
import jax.numpy as jnp
from jax import lax
from jax.experimental import pallas as pl
from jax.experimental.pallas import tpu as pltpu
from jax.experimental.pallas import tpu_sc as plsc

_NUM_IN = 20
_NUM_OUT = 64
_BATCH = 16384
_NW = 32
_BPW = _BATCH // _NW
_GRP = 16
_NGRP = _BPW // _GRP
_ELEMS = _BPW * _NUM_OUT
_ROW = 128
_NROW = _ELEMS // _ROW


def _sc_body(x_hbm, lut_hbm, out_hbm, xv, idxl, dst,
             sg0, sg1, sg2, sg3, so):
    wid = lax.axis_index("s") * 2 + lax.axis_index("c")
    for rb in range(3):
        pltpu.sync_copy(
            x_hbm.at[pl.ds((rb * 128 + 4 * wid) * 1024, 4096)],
            xv.at[pl.ds(rb * 4096, 4096)])

    def grp(g, carry):
        q = g >> 3
        col = (g & 7) * _GRP
        acc = jnp.zeros((_GRP,), jnp.int32)
        for i in range(_NUM_IN):
            v = xv[pl.ds((i // 8) * 4096 + q * 1024 + (i % 8) * 128 + col,
                         _GRP)]
            acc = acc + jnp.where(v > 0.0, jnp.int32(1 << i), jnp.int32(0))
        addr0 = ((acc >> 7) << 10) + (acc & 127)
        for c in range(_NUM_OUT):
            off = (c // 8) * 8388608 + (c % 8) * 128
            idxl[pl.ds((q * 64 + c) * _ROW + col, _GRP)] = addr0 + off

        sgs = (sg0, sg1, sg2, sg3)
        for qq in range(4):

            @pl.when(g == qq * 8 + 7)
            def _fire(qq=qq):
                pltpu.async_copy(
                    lut_hbm.at[idxl.at[pl.ds(qq * 8192, 8192)]],
                    dst.at[pl.ds(qq * 8192, 8192)], sgs[qq])

        return carry

    lax.fori_loop(0, _NGRP, grp, 0)
    for q, sg in enumerate((sg0, sg1, sg2, sg3)):
        pltpu.make_async_copy(lut_hbm.at[pl.ds(0, 8192)],
                              dst.at[pl.ds(q * 8192, 8192)], sg).wait()
        for cb8 in range(8):
            pltpu.async_copy(
                dst.at[pl.ds((q * 8 + cb8) * 1024, 1024)],
                out_hbm.at[pl.ds(cb8 * 131072 + (4 * wid + q) * 1024,
                                 1024)], so)
    pltpu.make_async_copy(dst, out_hbm.at[pl.ds(0, _ELEMS)], so).wait()


@functools.partial(jax.jit, static_argnames=())
def kernel(x, lut):
    mesh = plsc.VectorSubcoreMesh(core_axis_name="c", subcore_axis_name="s")
    run = pl.kernel(
        _sc_body,
        out_type=jax.ShapeDtypeStruct((_BATCH * _NUM_OUT,), jnp.float32),
        mesh=mesh,
        scratch_types=(
            pltpu.VMEM((3 * 4096,), jnp.float32),
            pltpu.VMEM((_ELEMS,), jnp.int32),
            pltpu.VMEM((_ELEMS,), jnp.float32),
            pltpu.SemaphoreType.DMA,
            pltpu.SemaphoreType.DMA,
            pltpu.SemaphoreType.DMA,
            pltpu.SemaphoreType.DMA,
            pltpu.SemaphoreType.DMA,
        ),
        compiler_params=pltpu.CompilerParams(
            needs_layout_passes=False, use_tc_tiling_on_sc=False),
    )
    lut_flat = (
        lut.T.reshape(8, 8, 8192, 128).transpose(0, 2, 1, 3).reshape(-1)
    )
    xp = jnp.concatenate(
        [x, jnp.zeros((_BATCH, 24 - _NUM_IN), jnp.float32)], axis=1)
    x_flat = xp.T.reshape(3, 8, 128, 128).transpose(0, 2, 1, 3).reshape(-1)
    out = run(x_flat, lut_flat)
    return out.reshape(8, 128, 8, 128).transpose(0, 2, 1, 3).reshape(
        _NUM_OUT, _BATCH).T


# speedup vs baseline: 1.2321x; 1.0001x over previous
import functools

import jax
import jax.numpy as jnp
from jax import lax
from jax.experimental import pallas as pl
from jax.experimental.pallas import tpu as pltpu
from jax.experimental.pallas import tpu_sc as plsc

_NUM_IN = 20
_NUM_OUT = 64
_BATCH = 16384
_NW = 32
_BPW = _BATCH // _NW
_GRP = 16
_NGRP = _BPW // _GRP
_ELEMS = _BPW * _NUM_OUT
_ROW = 128
_NROW = _ELEMS // _ROW


def _sc_body(x_hbm, lut_hbm, out_hbm, xv, idxl, dst,
             sg0, sg1, sg2, sg3, so):
    wid = lax.axis_index("s") * 2 + lax.axis_index("c")
    for rb in range(3):
        pltpu.sync_copy(
            x_hbm.at[pl.ds((rb * 128 + 4 * wid) * 1024, 4096)],
            xv.at[pl.ds(rb * 4096, 4096)])

    def grp(g, carry):
        q = g >> 3
        col = (g & 7) * _GRP
        acc = jnp.zeros((_GRP,), jnp.int32)
        for i in range(_NUM_IN):
            v = xv[pl.ds((i // 8) * 4096 + q * 1024 + (i % 8) * 128 + col,
                         _GRP)]
            acc = acc + jnp.where(v > 0.0, jnp.int32(1 << i), jnp.int32(0))
        addr0 = ((acc >> 7) << 10) + (acc & 127)
        def cexp(cb8, carry2):
            pos = (q * 64 + cb8 * 8) * _ROW + col
            a = addr0 + cb8 * 8388608
            for cr in range(8):
                idxl[pl.ds(pos + cr * _ROW, _GRP)] = a + cr * 128
            return carry2

        lax.fori_loop(0, 8, cexp, 0)

        sgs = (sg0, sg1, sg2, sg3)
        for qq in range(4):

            @pl.when(g == qq * 8 + 7)
            def _fire(qq=qq):
                pltpu.async_copy(
                    lut_hbm.at[idxl.at[pl.ds(qq * 8192, 8192)]],
                    dst.at[pl.ds(qq * 8192, 8192)], sgs[qq])

        return carry

    lax.fori_loop(0, _NGRP, grp, 0)
    for q, sg in enumerate((sg0, sg1, sg2, sg3)):
        pltpu.make_async_copy(lut_hbm.at[pl.ds(0, 8192)],
                              dst.at[pl.ds(q * 8192, 8192)], sg).wait()
        for cb8 in range(8):
            pltpu.async_copy(
                dst.at[pl.ds((q * 8 + cb8) * 1024, 1024)],
                out_hbm.at[pl.ds(cb8 * 131072 + (4 * wid + q) * 1024,
                                 1024)], so)
    pltpu.make_async_copy(dst, out_hbm.at[pl.ds(0, _ELEMS)], so).wait()


@functools.partial(jax.jit, static_argnames=())
def kernel(x, lut):
    mesh = plsc.VectorSubcoreMesh(core_axis_name="c", subcore_axis_name="s")
    run = pl.kernel(
        _sc_body,
        out_type=jax.ShapeDtypeStruct((_BATCH * _NUM_OUT,), jnp.float32),
        mesh=mesh,
        scratch_types=(
            pltpu.VMEM((3 * 4096,), jnp.float32),
            pltpu.VMEM((_ELEMS,), jnp.int32),
            pltpu.VMEM((_ELEMS,), jnp.float32),
            pltpu.SemaphoreType.DMA,
            pltpu.SemaphoreType.DMA,
            pltpu.SemaphoreType.DMA,
            pltpu.SemaphoreType.DMA,
            pltpu.SemaphoreType.DMA,
        ),
        compiler_params=pltpu.CompilerParams(
            needs_layout_passes=False, use_tc_tiling_on_sc=False),
    )
    lut_flat = (
        lut.T.reshape(8, 8, 8192, 128).transpose(0, 2, 1, 3).reshape(-1)
    )
    xp = jnp.concatenate(
        [x, jnp.zeros((_BATCH, 24 - _NUM_IN), jnp.float32)], axis=1)
    x_flat = xp.T.reshape(3, 8, 128, 128).transpose(0, 2, 1, 3).reshape(-1)
    out = run(x_flat, lut_flat)
    return out.reshape(8, 128, 8, 128).transpose(0, 2, 1, 3).reshape(
        _NUM_OUT, _BATCH).T


# speedup vs baseline: 1.2329x; 1.0007x over previous
import functools

import jax
import jax.numpy as jnp
from jax import lax
from jax.experimental import pallas as pl
from jax.experimental.pallas import tpu as pltpu
from jax.experimental.pallas import tpu_sc as plsc

_NUM_IN = 20
_NUM_OUT = 64
_BATCH = 16384
_NW = 32
_BPW = _BATCH // _NW
_GRP = 16
_NGRP = _BPW // _GRP
_ELEMS = _BPW * _NUM_OUT
_ROW = 128


def _sc_body(x_hbm, lut_hbm, out_hbm, xv, idxl, dst,
             sg0, sg1, sg2, sg3, so):
    wid = lax.axis_index("s") * 2 + lax.axis_index("c")
    for rb in range(3):
        pltpu.sync_copy(
            x_hbm.at[pl.ds((rb * 128 + 4 * wid) * 1024, 4096)],
            xv.at[pl.ds(rb * 4096, 4096)])

    def grp(g, carry):
        q = g >> 3
        col = (g & 7) * _GRP
        acc = jnp.zeros((_GRP,), jnp.int32)
        for i in range(_NUM_IN):
            v = xv[pl.ds((i // 8) * 4096 + q * 1024 + (i % 8) * 128 + col,
                         _GRP)]
            acc = acc + jnp.where(v > 0.0, jnp.int32(1 << i), jnp.int32(0))
        addr0 = ((acc >> 7) << 10) + (acc & 127)
        def cexp(cb8, carry2):
            pos = (q * 64 + cb8 * 8) * _ROW + col
            a = addr0 + cb8 * 8388608
            for cr in range(8):
                idxl[pl.ds(pos + cr * _ROW, _GRP)] = a + cr * 128
            return carry2

        lax.fori_loop(0, 8, cexp, 0)

        sgs = (sg0, sg1, sg2, sg3)
        for qq in range(4):

            @pl.when(g == qq * 8 + 7)
            def _fire(qq=qq):
                pltpu.async_copy(
                    lut_hbm.at[idxl.at[pl.ds(qq * 8192, 8192)]],
                    dst.at[pl.ds(qq * 8192, 8192)], sgs[qq])

        return carry

    lax.fori_loop(0, _NGRP, grp, 0)
    for q, sg in enumerate((sg0, sg1, sg2, sg3)):
        pltpu.make_async_copy(lut_hbm.at[pl.ds(0, 8192)],
                              dst.at[pl.ds(q * 8192, 8192)], sg).wait()
        for cb8 in range(8):
            pltpu.async_copy(
                dst.at[pl.ds((q * 8 + cb8) * 1024, 1024)],
                out_hbm.at[pl.ds(cb8 * 131072 + (4 * wid + q) * 1024,
                                 1024)], so)
    pltpu.make_async_copy(dst, out_hbm.at[pl.ds(0, _ELEMS)], so).wait()


@functools.partial(jax.jit, static_argnames=())
def kernel(x, lut):
    mesh = plsc.VectorSubcoreMesh(core_axis_name="c", subcore_axis_name="s")
    run = pl.kernel(
        _sc_body,
        out_type=jax.ShapeDtypeStruct((_BATCH * _NUM_OUT,), jnp.float32),
        mesh=mesh,
        scratch_types=(
            pltpu.VMEM((3 * 4096,), jnp.float32),
            pltpu.VMEM((_ELEMS,), jnp.int32),
            pltpu.VMEM((_ELEMS,), jnp.float32),
            pltpu.SemaphoreType.DMA,
            pltpu.SemaphoreType.DMA,
            pltpu.SemaphoreType.DMA,
            pltpu.SemaphoreType.DMA,
            pltpu.SemaphoreType.DMA,
        ),
        compiler_params=pltpu.CompilerParams(
            needs_layout_passes=False, use_tc_tiling_on_sc=False),
    )
    lut_flat = (
        lut.T.reshape(8, 8, 8192, 128).transpose(0, 2, 1, 3).reshape(-1)
    )
    xp = jnp.concatenate(
        [x, jnp.zeros((_BATCH, 24 - _NUM_IN), jnp.float32)], axis=1)
    x_flat = xp.T.reshape(3, 8, 128, 128).transpose(0, 2, 1, 3).reshape(-1)
    out = run(x_flat, lut_flat)
    return out.reshape(8, 128, 8, 128).transpose(0, 2, 1, 3).reshape(
        _NUM_OUT, _BATCH).T
